# baseline (speedup 1.0000x reference)
import jax
import jax.numpy as jnp
from jax import lax
from jax.experimental import pallas as pl
from jax.experimental.pallas import tpu as pltpu
from jax.experimental.pallas import tpu_sc as plsc

TOT = 1048576
NRAY = 16384
CJ = 128
RA = TOT // CJ
MS = RA // 128
L = 128

NW = 32
RPW = NRAY // NW
FPW = TOT // NW


def _lane_roll(v, s):
    if s == 0:
        return v
    return jnp.concatenate([v[:, -s:], v[:, :-s]], axis=1)


def _sub_roll(v, s):
    if s == 0:
        return v
    return jnp.concatenate([v[-s:], v[:-s]], axis=0)


def _flat_shift(v, s, fill, lane, flat_r):
    ln, sb = s % L, s // L
    if ln:
        x = _lane_roll(v, ln)
        x = jnp.where(lane < ln, _sub_roll(x, sb + 1), _sub_roll(x, sb))
    else:
        x = _sub_roll(v, sb)
    return jnp.where(flat_r < s, fill, x)


def _flat_shift_b(v, s, fill, lane, flat_r):
    ln, sb = s % L, s // L
    if ln:
        x = _lane_roll(v, L - ln)
        x = jnp.where(lane >= L - ln, _sub_roll(x, MS - sb - 1),
                      _sub_roll(x, (MS - sb) % MS))
    else:
        x = _sub_roll(v, (MS - sb) % MS)
    return jnp.where(flat_r >= MS * L - s, fill, x)


def _tc_main(aT_ref, r0_ref, r1_ref, r2_ref, f_ref,
             trans_ref, lomc_ref, wc_ref, wr_ref, wg_ref, wb_ref,
             i0, aT, fpT):
    subl = lax.broadcasted_iota(jnp.int32, (MS, L), 0)
    lane = lax.broadcasted_iota(jnp.int32, (MS, L), 1)
    flat_r = subl * L + lane

    aTv = jnp.transpose(aT_ref[...]).reshape(CJ, MS, L)
    lomc_ref[...] = jnp.log(jnp.clip(1.0 - aTv, 1e-7, 1.0))

    i0[...] = lomc_ref[...]

    def scan_body(j, prev):
        cur = i0[j] + prev
        i0[j] = cur
        return cur

    lax.fori_loop(1, CJ, scan_body, i0[0], unroll=8)

    s0 = i0[CJ - 1]
    def l1_body(l, acc):
        sh = _lane_roll(acc, 1)
        return jnp.where(lane == l, acc + sh, acc)

    i1 = lax.fori_loop(1, L, l1_body, s0, unroll=8)

    lcol = i1[:, L - 1:L]
    subc = lax.broadcasted_iota(jnp.int32, (MS, 1), 0)
    y = jnp.zeros((MS, 1), jnp.float32)

    def l2_body(m, y):
        return jnp.where(subc == m, _sub_roll(y, 1) + _sub_roll(lcol, 1), y)

    y = lax.fori_loop(1, MS, l2_body, y, unroll=8)

    s1 = i1 + y
    a_ln = _lane_roll(s1, 1)
    b_sl = _sub_roll(a_ln, 1)
    off = jnp.where(lane == 0, b_sl, a_ln)
    off = jnp.where((lane == 0) & (subl == 0), 0.0, off)

    i0[...] = (i0[...] + off[None]) - lomc_ref[...]

    a0 = jnp.where(f_ref[0] == 1, i0[0], 0.0)
    aT[0] = a0
    fpT[0] = f_ref[0]

    def prop_body(j, cr):
        ap, fpp = cr
        fj = f_ref[j]
        aj = jnp.where(fj == 1, i0[j], ap)
        fpj = fpp | fj
        aT[j] = aj
        fpT[j] = fpj
        return (aj, fpj)

    lax.fori_loop(1, CJ, prop_body, (a0, f_ref[0]), unroll=8)

    rowFA = fpT[CJ - 1]
    val, seen = aT[CJ - 1], rowFA
    s = 1
    while s < MS * L:
        vs = _flat_shift(val, s, 0.0, lane, flat_r)
        ss = _flat_shift(seen, s, 0, lane, flat_r)
        val = jnp.where(seen == 1, val, vs)
        seen = seen | ss
        s *= 2
    rowcarry = _flat_shift(val, 1, 0.0, lane, flat_r)

    fp_full = fpT[...]
    afull = jnp.where(fp_full == 1, aT[...], rowcarry[None])

    transT = jnp.exp(i0[...] - afull)
    alphaT = 1.0 - jnp.exp(lomc_ref[...])
    w = alphaT * transT
    trans_ref[...] = jnp.transpose(transT.reshape(CJ, RA))

    wc_ref[...] = w
    wr_ref[...] = w * r0_ref[...]
    wg_ref[...] = w * r1_ref[...]
    wb_ref[...] = w * r2_ref[...]

    refs = (lomc_ref, wc_ref, wr_ref, wg_ref, wb_ref)

    def seg_body(j, cr):
        fj = f_ref[j] == 1
        out = []
        for ref, c_prev in zip(refs, cr):
            cur = ref[j] + jnp.where(fj, 0.0, c_prev)
            ref[j] = cur
            out.append(cur)
        return tuple(out)

    init = tuple(ref[0] for ref in refs)
    lax.fori_loop(1, CJ, seg_body, init, unroll=8)

    for ref in refs:
        S = ref[CJ - 1]
        FAv = rowFA
        s = 1
        while s < MS * L:
            Ss = _flat_shift(S, s, 0.0, lane, flat_r)
            Fs = _flat_shift(FAv, s, 0, lane, flat_r)
            S = jnp.where(FAv == 1, S, S + Ss)
            FAv = FAv | Fs
            s *= 2
        rowc = _flat_shift(S, 1, 0.0, lane, flat_r)
        ref[...] = ref[...] + jnp.where(fp_full == 1, 0.0, rowc[None])


def _tc_back(wc_ref, f_ref, wsps_ref, gT, bpT):
    subl = lax.broadcasted_iota(jnp.int32, (MS, L), 0)
    lane = lax.broadcasted_iota(jnp.int32, (MS, L), 1)
    flat_r = subl * L + lane

    fn_last = _flat_shift_b(f_ref[0], 1, 1, lane, flat_r)

    g127 = jnp.where(fn_last == 1, wc_ref[CJ - 1], 0.0)
    gT[CJ - 1] = g127
    bpT[CJ - 1] = fn_last

    def back_body(jj, cr):
        gp, bpp = cr
        j = CJ - 2 - jj
        fnj = f_ref[j + 1]
        gj = jnp.where(fnj == 1, wc_ref[j], gp)
        bpj = bpp | fnj
        gT[j] = gj
        bpT[j] = bpj
        return (gj, bpj)

    lax.fori_loop(0, CJ - 1, back_body, (g127, fn_last), unroll=8)

    val, seen = gT[0], bpT[0]
    s = 1
    while s < MS * L:
        vs = _flat_shift_b(val, s, 0.0, lane, flat_r)
        ss = _flat_shift_b(seen, s, 0, lane, flat_r)
        val = jnp.where(seen == 1, val, vs)
        seen = seen | ss
        s *= 2
    rowcarry = _flat_shift_b(val, 1, 0.0, lane, flat_r)

    gfull = jnp.where(bpT[...] == 1, gT[...], rowcarry[None])
    wsps_ref[...] = jnp.transpose(gfull.reshape(CJ, RA))


def _flat_t_vec(e):
    return ((e & (CJ - 1)) << 13) + (e >> 7)


def _sc_wid():
    return lax.axis_index("s") * 2 + lax.axis_index("c")


def _sc_scatter_flags(cu_hbm, f_hbm, cuv, fbuf):
    wid = _sc_wid()
    base = wid * FPW
    pltpu.sync_copy(cu_hbm.at[pl.ds(0, NRAY)], cuv)
    zero16 = jnp.zeros((16,), jnp.int32)

    def zbody(i, _):
        fbuf[pl.ds(pl.multiple_of(i * 16, 16), 16)] = zero16
        return 0

    lax.fori_loop(0, FPW // 16, zbody, 0, unroll=8)
    one16 = jnp.full((16,), 1, jnp.int32)

    def sbody(t, _):
        st = cuv[pl.ds(pl.multiple_of(t * 16, 16), 16)]
        fe = _flat_t_vec(st)
        loc = fe - base
        mask = (fe >= base) & (fe < base + FPW)
        plsc.store_scatter(fbuf, [jnp.where(mask, loc, 0)], one16, mask=mask)
        return 0

    lax.fori_loop(0, NRAY // 16, sbody, 0, unroll=8)
    pltpu.sync_copy(fbuf, f_hbm.at[pl.ds(base, FPW)])


def _sc_ray_outputs(cu_hbm, lomc_hbm, wc_hbm, wr_hbm, wg_hbm, wb_hbm,
                    bg_hbm, wspr_hbm, ir_hbm, ig_hbm, ib_hbm,
                    cuv, cuv1, idxv, vv, g0, g1, g2, g3, g4, sem):
    wid = _sc_wid()
    base = wid * RPW
    pltpu.sync_copy(cu_hbm.at[pl.ds(base, RPW)], cuv)
    pltpu.sync_copy(cu_hbm.at[pl.ds(base + 8, RPW)], cuv1)
    lanes = jax.lax.iota(jnp.int32, 16)

    for t in range(RPW // 16):
        sl = pl.ds(t * 16, 16)
        st = cuv[sl]
        nxt = cuv1[sl]
        rot = st.at[(lanes + 1) & 15].get(mode="promise_in_bounds")
        b8 = nxt.at[jnp.full((16,), 8, jnp.int32)].get(mode="promise_in_bounds")
        en = jnp.where(lanes < 15, rot, b8)
        valid = en > st
        e = jnp.maximum(en - 1, 0)
        idxv[sl] = _flat_t_vec(e)
        vv[sl] = jnp.where(valid, 1, 0)

    c0 = pltpu.async_copy(lomc_hbm.at[idxv], g0, sem)
    c1 = pltpu.async_copy(wc_hbm.at[idxv], g1, sem)
    c2 = pltpu.async_copy(wr_hbm.at[idxv], g2, sem)
    c3 = pltpu.async_copy(wg_hbm.at[idxv], g3, sem)
    c4 = pltpu.async_copy(wb_hbm.at[idxv], g4, sem)
    c0.wait(); c1.wait(); c2.wait(); c3.wait(); c4.wait()

    for t in range(RPW // 16):
        sl = pl.ds(t * 16, 16)
        v = vv[sl] == 1
        zero = jnp.zeros((16,), jnp.float32)
        g0[sl] = jnp.exp(jnp.where(v, g0[sl], zero))
        g1[sl] = jnp.where(v, g1[sl], zero)
        g2[sl] = jnp.where(v, g2[sl], zero)
        g3[sl] = jnp.where(v, g3[sl], zero)
        g4[sl] = jnp.where(v, g4[sl], zero)

    pltpu.sync_copy(g0, bg_hbm.at[pl.ds(base, RPW)])
    pltpu.sync_copy(g1, wspr_hbm.at[pl.ds(base, RPW)])
    pltpu.sync_copy(g2, ir_hbm.at[pl.ds(base, RPW)])
    pltpu.sync_copy(g3, ig_hbm.at[pl.ds(base, RPW)])
    pltpu.sync_copy(g4, ib_hbm.at[pl.ds(base, RPW)])


@jax.jit
def kernel(cu_seqlens, alpha, rgb):
    cu = cu_seqlens
    mesh = plsc.VectorSubcoreMesh(core_axis_name="c", subcore_axis_name="s")

    fT = pl.kernel(
        _sc_scatter_flags, mesh=mesh,
        out_type=jax.ShapeDtypeStruct((TOT,), jnp.int32),
        scratch_types=[pltpu.VMEM((NRAY,), jnp.int32),
                       pltpu.VMEM((FPW,), jnp.int32)],
        compiler_params=pltpu.CompilerParams(needs_layout_passes=False),
    )(cu)
    fT = fT.reshape(CJ, MS, L)

    aT = alpha.reshape(RA, CJ)
    rgbT = jnp.transpose(rgb.reshape(RA, CJ, 3), (2, 1, 0)).reshape(3, CJ, MS, L)

    out_t = jax.ShapeDtypeStruct((CJ, MS, L), jnp.float32)
    trans, lomc, wc, wr, wg, wb = pl.pallas_call(
        _tc_main,
        out_shape=[jax.ShapeDtypeStruct((RA, CJ), jnp.float32),
                   out_t, out_t, out_t, out_t, out_t],
        scratch_shapes=[pltpu.VMEM((CJ, MS, L), jnp.float32),
                        pltpu.VMEM((CJ, MS, L), jnp.float32),
                        pltpu.VMEM((CJ, MS, L), jnp.int32)],
        compiler_params=pltpu.CompilerParams(
            vmem_limit_bytes=100 * 1024 * 1024),
    )(aT, rgbT[0], rgbT[1], rgbT[2], fT)

    wsps = pl.pallas_call(
        _tc_back,
        out_shape=jax.ShapeDtypeStruct((RA, CJ), jnp.float32),
        scratch_shapes=[pltpu.VMEM((CJ, MS, L), jnp.float32),
                        pltpu.VMEM((CJ, MS, L), jnp.int32)],
    )(wc, fT)

    cu_p = jnp.concatenate([cu, jnp.zeros((7,), cu.dtype)])
    ray_out = jax.ShapeDtypeStruct((NRAY,), jnp.float32)
    bg, wspr, ir, ig, ib = pl.kernel(
        _sc_ray_outputs, mesh=mesh,
        out_type=[ray_out] * 5,
        scratch_types=[pltpu.VMEM((RPW,), jnp.int32),
                       pltpu.VMEM((RPW,), jnp.int32),
                       pltpu.VMEM((RPW,), jnp.int32),
                       pltpu.VMEM((RPW,), jnp.int32),
                       pltpu.VMEM((RPW,), jnp.float32),
                       pltpu.VMEM((RPW,), jnp.float32),
                       pltpu.VMEM((RPW,), jnp.float32),
                       pltpu.VMEM((RPW,), jnp.float32),
                       pltpu.VMEM((RPW,), jnp.float32),
                       pltpu.SemaphoreType.DMA],
        compiler_params=pltpu.CompilerParams(needs_layout_passes=False),
    )(cu_p, lomc.reshape(-1), wc.reshape(-1), wr.reshape(-1),
      wg.reshape(-1), wb.reshape(-1))
    irgb = jnp.stack([ir, ig, ib], axis=1)

    return (trans.reshape(TOT, 1), bg[:, None], irgb, wspr[:, None],
            wsps.reshape(TOT, 1))


# speedup vs baseline: 1.0140x; 1.0140x over previous
import jax
import jax.numpy as jnp
from jax import lax
from jax.experimental import pallas as pl
from jax.experimental.pallas import tpu as pltpu
from jax.experimental.pallas import tpu_sc as plsc

TOT = 1048576
NRAY = 16384
CJ = 128
RA = TOT // CJ
MS = RA // 128
L = 128

NW = 32
RPW = NRAY // NW
FPW = TOT // NW


def _lane_roll(v, s):
    if s == 0:
        return v
    return jnp.concatenate([v[:, -s:], v[:, :-s]], axis=1)


def _sub_roll(v, s):
    if s == 0:
        return v
    return jnp.concatenate([v[-s:], v[:-s]], axis=0)


def _flat_shift(v, s, fill, lane, flat_r):
    ln, sb = s % L, s // L
    if ln:
        x = _lane_roll(v, ln)
        x = jnp.where(lane < ln, _sub_roll(x, sb + 1), _sub_roll(x, sb))
    else:
        x = _sub_roll(v, sb)
    return jnp.where(flat_r < s, fill, x)


def _flat_shift_b(v, s, fill, lane, flat_r):
    ln, sb = s % L, s // L
    if ln:
        x = _lane_roll(v, L - ln)
        x = jnp.where(lane >= L - ln, _sub_roll(x, MS - sb - 1),
                      _sub_roll(x, (MS - sb) % MS))
    else:
        x = _sub_roll(v, (MS - sb) % MS)
    return jnp.where(flat_r >= MS * L - s, fill, x)


def _tc_main(aT_ref, r0_ref, r1_ref, r2_ref, f_ref,
             trans_ref, lomc_ref, wc_ref, wr_ref, wg_ref, wb_ref,
             i0, aT, fpT):
    subl = lax.broadcasted_iota(jnp.int32, (MS, L), 0)
    lane = lax.broadcasted_iota(jnp.int32, (MS, L), 1)
    flat_r = subl * L + lane

    lomc_ref[...] = jnp.log(jnp.clip(1.0 - aT_ref[...], 1e-7, 1.0))

    i0[...] = lomc_ref[...]

    def scan_body(j, prev):
        cur = i0[j] + prev
        i0[j] = cur
        return cur

    lax.fori_loop(1, CJ, scan_body, i0[0], unroll=8)

    s0 = i0[CJ - 1]
    def l1_body(l, acc):
        sh = _lane_roll(acc, 1)
        return jnp.where(lane == l, acc + sh, acc)

    i1 = lax.fori_loop(1, L, l1_body, s0, unroll=8)

    lcol = i1[:, L - 1:L]
    subc = lax.broadcasted_iota(jnp.int32, (MS, 1), 0)
    y = jnp.zeros((MS, 1), jnp.float32)

    def l2_body(m, y):
        return jnp.where(subc == m, _sub_roll(y, 1) + _sub_roll(lcol, 1), y)

    y = lax.fori_loop(1, MS, l2_body, y, unroll=8)

    s1 = i1 + y
    a_ln = _lane_roll(s1, 1)
    b_sl = _sub_roll(a_ln, 1)
    off = jnp.where(lane == 0, b_sl, a_ln)
    off = jnp.where((lane == 0) & (subl == 0), 0.0, off)

    i0[...] = (i0[...] + off[None]) - lomc_ref[...]

    a0 = jnp.where(f_ref[0] == 1, i0[0], 0.0)
    aT[0] = a0
    fpT[0] = f_ref[0]

    def prop_body(j, cr):
        ap, fpp = cr
        fj = f_ref[j]
        aj = jnp.where(fj == 1, i0[j], ap)
        fpj = fpp | fj
        aT[j] = aj
        fpT[j] = fpj
        return (aj, fpj)

    lax.fori_loop(1, CJ, prop_body, (a0, f_ref[0]), unroll=8)

    rowFA = fpT[CJ - 1]
    val, seen = aT[CJ - 1], rowFA
    s = 1
    while s < MS * L:
        vs = _flat_shift(val, s, 0.0, lane, flat_r)
        ss = _flat_shift(seen, s, 0, lane, flat_r)
        val = jnp.where(seen == 1, val, vs)
        seen = seen | ss
        s *= 2
    rowcarry = _flat_shift(val, 1, 0.0, lane, flat_r)

    fp_full = fpT[...]
    afull = jnp.where(fp_full == 1, aT[...], rowcarry[None])

    transT = jnp.exp(i0[...] - afull)
    alphaT = 1.0 - jnp.exp(lomc_ref[...])
    w = alphaT * transT
    trans_ref[...] = jnp.transpose(transT.reshape(CJ, RA))

    wc_ref[...] = w
    wr_ref[...] = w * r0_ref[...]
    wg_ref[...] = w * r1_ref[...]
    wb_ref[...] = w * r2_ref[...]

    refs = (lomc_ref, wc_ref, wr_ref, wg_ref, wb_ref)

    def seg_body(j, cr):
        fj = f_ref[j] == 1
        out = []
        for ref, c_prev in zip(refs, cr):
            cur = ref[j] + jnp.where(fj, 0.0, c_prev)
            ref[j] = cur
            out.append(cur)
        return tuple(out)

    init = tuple(ref[0] for ref in refs)
    lax.fori_loop(1, CJ, seg_body, init, unroll=8)

    for ref in refs:
        S = ref[CJ - 1]
        FAv = rowFA
        s = 1
        while s < MS * L:
            Ss = _flat_shift(S, s, 0.0, lane, flat_r)
            Fs = _flat_shift(FAv, s, 0, lane, flat_r)
            S = jnp.where(FAv == 1, S, S + Ss)
            FAv = FAv | Fs
            s *= 2
        rowc = _flat_shift(S, 1, 0.0, lane, flat_r)
        ref[...] = ref[...] + jnp.where(fp_full == 1, 0.0, rowc[None])


def _tc_back(wc_ref, f_ref, wsps_ref, gT, bpT):
    subl = lax.broadcasted_iota(jnp.int32, (MS, L), 0)
    lane = lax.broadcasted_iota(jnp.int32, (MS, L), 1)
    flat_r = subl * L + lane

    fn_last = _flat_shift_b(f_ref[0], 1, 1, lane, flat_r)

    g127 = jnp.where(fn_last == 1, wc_ref[CJ - 1], 0.0)
    gT[CJ - 1] = g127
    bpT[CJ - 1] = fn_last

    def back_body(jj, cr):
        gp, bpp = cr
        j = CJ - 2 - jj
        fnj = f_ref[j + 1]
        gj = jnp.where(fnj == 1, wc_ref[j], gp)
        bpj = bpp | fnj
        gT[j] = gj
        bpT[j] = bpj
        return (gj, bpj)

    lax.fori_loop(0, CJ - 1, back_body, (g127, fn_last), unroll=8)

    val, seen = gT[0], bpT[0]
    s = 1
    while s < MS * L:
        vs = _flat_shift_b(val, s, 0.0, lane, flat_r)
        ss = _flat_shift_b(seen, s, 0, lane, flat_r)
        val = jnp.where(seen == 1, val, vs)
        seen = seen | ss
        s *= 2
    rowcarry = _flat_shift_b(val, 1, 0.0, lane, flat_r)

    gfull = jnp.where(bpT[...] == 1, gT[...], rowcarry[None])
    wsps_ref[...] = jnp.transpose(gfull.reshape(CJ, RA))


def _flat_t_vec(e):
    return ((e & (CJ - 1)) << 13) + (e >> 7)


def _sc_wid():
    return lax.axis_index("s") * 2 + lax.axis_index("c")


def _sc_scatter_flags(cu_hbm, f_hbm, cuv, fbuf):
    wid = _sc_wid()
    base = wid * FPW
    pltpu.sync_copy(cu_hbm.at[pl.ds(0, NRAY)], cuv)
    zero16 = jnp.zeros((16,), jnp.int32)

    def zbody(i, _):
        fbuf[pl.ds(pl.multiple_of(i * 16, 16), 16)] = zero16
        return 0

    lax.fori_loop(0, FPW // 16, zbody, 0, unroll=8)
    one16 = jnp.full((16,), 1, jnp.int32)

    def sbody(t, _):
        st = cuv[pl.ds(pl.multiple_of(t * 16, 16), 16)]
        fe = _flat_t_vec(st)
        loc = fe - base
        mask = (fe >= base) & (fe < base + FPW)
        plsc.store_scatter(fbuf, [jnp.where(mask, loc, 0)], one16, mask=mask)
        return 0

    lax.fori_loop(0, NRAY // 16, sbody, 0, unroll=8)
    pltpu.sync_copy(fbuf, f_hbm.at[pl.ds(base, FPW)])


def _sc_ray_outputs(cu_hbm, lomc_hbm, wc_hbm, wr_hbm, wg_hbm, wb_hbm,
                    bg_hbm, wspr_hbm, ir_hbm, ig_hbm, ib_hbm,
                    cuv, cuv1, idxv, vv, g0, g1, g2, g3, g4, sem):
    wid = _sc_wid()
    base = wid * RPW
    pltpu.sync_copy(cu_hbm.at[pl.ds(base, RPW)], cuv)
    pltpu.sync_copy(cu_hbm.at[pl.ds(base + 8, RPW)], cuv1)
    lanes = jax.lax.iota(jnp.int32, 16)

    for t in range(RPW // 16):
        sl = pl.ds(t * 16, 16)
        st = cuv[sl]
        nxt = cuv1[sl]
        rot = st.at[(lanes + 1) & 15].get(mode="promise_in_bounds")
        b8 = nxt.at[jnp.full((16,), 8, jnp.int32)].get(mode="promise_in_bounds")
        en = jnp.where(lanes < 15, rot, b8)
        valid = en > st
        e = jnp.maximum(en - 1, 0)
        idxv[sl] = _flat_t_vec(e)
        vv[sl] = jnp.where(valid, 1, 0)

    c0 = pltpu.async_copy(lomc_hbm.at[idxv], g0, sem)
    c1 = pltpu.async_copy(wc_hbm.at[idxv], g1, sem)
    c2 = pltpu.async_copy(wr_hbm.at[idxv], g2, sem)
    c3 = pltpu.async_copy(wg_hbm.at[idxv], g3, sem)
    c4 = pltpu.async_copy(wb_hbm.at[idxv], g4, sem)
    c0.wait(); c1.wait(); c2.wait(); c3.wait(); c4.wait()

    for t in range(RPW // 16):
        sl = pl.ds(t * 16, 16)
        v = vv[sl] == 1
        zero = jnp.zeros((16,), jnp.float32)
        g0[sl] = jnp.exp(jnp.where(v, g0[sl], zero))
        g1[sl] = jnp.where(v, g1[sl], zero)
        g2[sl] = jnp.where(v, g2[sl], zero)
        g3[sl] = jnp.where(v, g3[sl], zero)
        g4[sl] = jnp.where(v, g4[sl], zero)

    pltpu.sync_copy(g0, bg_hbm.at[pl.ds(base, RPW)])
    pltpu.sync_copy(g1, wspr_hbm.at[pl.ds(base, RPW)])
    pltpu.sync_copy(g2, ir_hbm.at[pl.ds(base, RPW)])
    pltpu.sync_copy(g3, ig_hbm.at[pl.ds(base, RPW)])
    pltpu.sync_copy(g4, ib_hbm.at[pl.ds(base, RPW)])


@jax.jit
def kernel(cu_seqlens, alpha, rgb):
    cu = cu_seqlens
    mesh = plsc.VectorSubcoreMesh(core_axis_name="c", subcore_axis_name="s")

    fT = pl.kernel(
        _sc_scatter_flags, mesh=mesh,
        out_type=jax.ShapeDtypeStruct((TOT,), jnp.int32),
        scratch_types=[pltpu.VMEM((NRAY,), jnp.int32),
                       pltpu.VMEM((FPW,), jnp.int32)],
        compiler_params=pltpu.CompilerParams(needs_layout_passes=False),
    )(cu)
    fT = fT.reshape(CJ, MS, L)

    aT = jnp.transpose(alpha.reshape(RA, CJ)).reshape(CJ, MS, L)
    rgbT = jnp.transpose(rgb.reshape(RA, CJ, 3), (2, 1, 0)).reshape(3, CJ, MS, L)

    out_t = jax.ShapeDtypeStruct((CJ, MS, L), jnp.float32)
    trans, lomc, wc, wr, wg, wb = pl.pallas_call(
        _tc_main,
        out_shape=[jax.ShapeDtypeStruct((RA, CJ), jnp.float32),
                   out_t, out_t, out_t, out_t, out_t],
        scratch_shapes=[pltpu.VMEM((CJ, MS, L), jnp.float32),
                        pltpu.VMEM((CJ, MS, L), jnp.float32),
                        pltpu.VMEM((CJ, MS, L), jnp.int32)],
        compiler_params=pltpu.CompilerParams(
            vmem_limit_bytes=100 * 1024 * 1024),
    )(aT, rgbT[0], rgbT[1], rgbT[2], fT)

    wsps = pl.pallas_call(
        _tc_back,
        out_shape=jax.ShapeDtypeStruct((RA, CJ), jnp.float32),
        scratch_shapes=[pltpu.VMEM((CJ, MS, L), jnp.float32),
                        pltpu.VMEM((CJ, MS, L), jnp.int32)],
    )(wc, fT)

    cu_p = jnp.concatenate([cu, jnp.zeros((7,), cu.dtype)])
    ray_out = jax.ShapeDtypeStruct((NRAY,), jnp.float32)
    bg, wspr, ir, ig, ib = pl.kernel(
        _sc_ray_outputs, mesh=mesh,
        out_type=[ray_out] * 5,
        scratch_types=[pltpu.VMEM((RPW,), jnp.int32),
                       pltpu.VMEM((RPW,), jnp.int32),
                       pltpu.VMEM((RPW,), jnp.int32),
                       pltpu.VMEM((RPW,), jnp.int32),
                       pltpu.VMEM((RPW,), jnp.float32),
                       pltpu.VMEM((RPW,), jnp.float32),
                       pltpu.VMEM((RPW,), jnp.float32),
                       pltpu.VMEM((RPW,), jnp.float32),
                       pltpu.VMEM((RPW,), jnp.float32),
                       pltpu.SemaphoreType.DMA],
        compiler_params=pltpu.CompilerParams(needs_layout_passes=False),
    )(cu_p, lomc.reshape(-1), wc.reshape(-1), wr.reshape(-1),
      wg.reshape(-1), wb.reshape(-1))
    irgb = jnp.stack([ir, ig, ib], axis=1)

    return (trans.reshape(TOT, 1), bg[:, None], irgb, wspr[:, None],
            wsps.reshape(TOT, 1))
